# Initial kernel scaffold; baseline (speedup 1.0000x reference)
import jax
import jax.numpy as jnp
from jax.experimental import pallas as pl


def kernel(word_ids, modalities_ids, age_ids, delays_ids, seg_ids, posi_ids, NPI_ids, word_table, modalities_table, seg_table, NPI_table, posi_table, age_table, delay_table, ln_gamma, ln_beta):
    raise NotImplementedError("write your pallas kernel here")


# speedup vs baseline: 1.0081x; 1.0081x over previous
import functools

import jax
import jax.numpy as jnp
from jax import lax
from jax.experimental import pallas as pl
from jax.experimental.pallas import tpu as pltpu
from jax.experimental.pallas import tpu_sc as plsc

H = 128
B = 1024
L = 200
BL = B * L
EPS = 1e-12

NC = 2
NS = 16
NW = NC * NS
TOK_PER_W = BL // NW
C = 128
N_CHUNKS = TOK_PER_W // C
NV = H // 16


def _rsqrt(x):
    i = lax.bitcast_convert_type(x, jnp.int32)
    i = jnp.int32(0x5F3759DF) - lax.shift_right_logical(i, 1)
    y = lax.bitcast_convert_type(i, jnp.float32)
    for _ in range(3):
        y = y * (1.5 - 0.5 * x * y * y)
    return y


_GDN = lax.GatherDimensionNumbers(
    offset_dims=(), collapsed_slice_dims=(0,), start_index_map=(0,))


def _perm(v, idx):
    return lax.gather(v, idx[:, None], _GDN, (1,),
                      mode=lax.GatherScatterMode.PROMISE_IN_BOUNDS)


def _hsum(v):
    idx = lax.iota(jnp.int32, 16)
    for d in (8, 4, 2, 1):
        v = v + _perm(v, idx ^ d)
    return v


_MESH = plsc.VectorSubcoreMesh(
    core_axis_name="c", subcore_axis_name="s", num_cores=NC, num_subcores=NS
)


@functools.partial(
    pl.kernel,
    out_type=jax.ShapeDtypeStruct((BL, H), jnp.float32),
    mesh=_MESH,
    scratch_types=(
        [pltpu.VMEM((C,), jnp.int32) for _ in range(7)]
        + [pltpu.VMEM((C, H), jnp.float32) for _ in range(7)]
        + [pltpu.VMEM((H,), jnp.float32), pltpu.VMEM((H,), jnp.float32),
           pltpu.SemaphoreType.DMA]
    ),
)
def _embed_ln(wi, mi, ai, di, si, pi, ni,
              wt, mt, st, nt, pt, at, dt, g, b,
              out,
              xw, xm, xa, xd, xs, xp, xn,
              rw, rm, ra, rd, rs, rp, rn,
              gv, bv, sem):
    wid = lax.axis_index("c") * NS + lax.axis_index("s")
    tok0 = wid * TOK_PER_W
    pltpu.sync_copy(g, gv)
    pltpu.sync_copy(b, bv)
    gs = [gv[pl.ds(k * 16, 16)] for k in range(NV)]
    bs = [bv[pl.ds(k * 16, 16)] for k in range(NV)]

    id_refs = (wi, mi, ai, di, si, pi, ni)
    idx_refs = (xw, xm, xa, xd, xs, xp, xn)
    tab_refs = (wt, mt, at, dt, st, pt, nt)
    row_refs = (rw, rm, ra, rd, rs, rp, rn)

    def chunk_body(ci, carry):
        base = tok0 + ci * C
        for ids, idx in zip(id_refs, idx_refs):
            pltpu.sync_copy(ids.at[pl.ds(base, C)], idx)
        handles = [pltpu.async_copy(tab.at[idx], rows, sem)
                   for tab, idx, rows in zip(tab_refs, idx_refs, row_refs)]
        for h in handles:
            h.wait()

        def tok_body(t, c2):
            vs = []
            for k in range(NV):
                sl = pl.ds(k * 16, 16)
                v = ((rw[t, sl] + rm[t, sl]) + (ra[t, sl] + rd[t, sl])
                     + ((rs[t, sl] + rp[t, sl]) + rn[t, sl]))
                vs.append(v)
            s = ((vs[0] + vs[1]) + (vs[2] + vs[3])) + (
                (vs[4] + vs[5]) + (vs[6] + vs[7]))
            sq = ((vs[0] * vs[0] + vs[1] * vs[1])
                  + (vs[2] * vs[2] + vs[3] * vs[3])) + (
                 (vs[4] * vs[4] + vs[5] * vs[5])
                  + (vs[6] * vs[6] + vs[7] * vs[7]))
            u = _hsum(s) * (1.0 / H)
            ex2 = _hsum(sq) * (1.0 / H)
            var = jnp.maximum(ex2 - u * u, 0.0)
            inv = _rsqrt(var + EPS)
            for k in range(NV):
                rw[t, pl.ds(k * 16, 16)] = (vs[k] - u) * inv * gs[k] + bs[k]
            return c2
        lax.fori_loop(0, C, tok_body, 0)
        pltpu.sync_copy(rw, out.at[pl.ds(base, C)])
        return carry

    lax.fori_loop(0, N_CHUNKS, chunk_body, 0)


def kernel(word_ids, modalities_ids, age_ids, delays_ids, seg_ids, posi_ids,
           NPI_ids, word_table, modalities_table, seg_table, NPI_table,
           posi_table, age_table, delay_table, ln_gamma, ln_beta):
    out = _embed_ln(
        word_ids.reshape(-1), modalities_ids.reshape(-1),
        age_ids.reshape(-1), delays_ids.reshape(-1),
        seg_ids.reshape(-1), posi_ids.reshape(-1), NPI_ids.reshape(-1),
        word_table, modalities_table, seg_table, NPI_table,
        posi_table, age_table, delay_table, ln_gamma, ln_beta)
    return out.reshape(B, L, H)
